# baseline (speedup 1.0000x reference)
import functools

import jax
import jax.numpy as jnp
from jax import lax
from jax.experimental import pallas as pl
from jax.experimental.pallas import tpu as pltpu
from jax.experimental.pallas import tpu_sc as plsc

_NC = 2
_NS = 16
_NW = _NC * _NS
_R = 32


def _sc_body(seq, pe, out, buf, pbuf, *, batch, seq_len, d):
    w = lax.axis_index("c") * _NS + lax.axis_index("s")
    s_per_w = seq_len // _NW
    s0 = w * s_per_w
    nvec = d // 16

    def chunk(i, carry):
        base = s0 + i * _R
        pltpu.sync_copy(pe.at[pl.ds(base, _R)], pbuf)
        for b in range(batch):
            row0 = b * seq_len + base
            pltpu.sync_copy(seq.at[pl.ds(row0, _R)], buf)

            def add_row(r, c2):
                for c in range(nvec):
                    sl = pl.ds(c * 16, 16)
                    buf[r, sl] = buf[r, sl] + pbuf[r, sl]
                return c2

            lax.fori_loop(0, _R, add_row, 0)
            pltpu.sync_copy(buf, out.at[pl.ds(row0, _R)])
        return carry

    lax.fori_loop(0, s_per_w // _R, chunk, 0)


def kernel(sequence, pe):
    b, s, d = sequence.shape
    rows = b * s
    seq2d = sequence.reshape(rows, d)
    mesh = plsc.VectorSubcoreMesh(
        core_axis_name="c", subcore_axis_name="s",
        num_cores=_NC, num_subcores=_NS,
    )
    body = functools.partial(_sc_body, batch=b, seq_len=s, d=d)
    out2d = pl.kernel(
        body,
        out_type=jax.ShapeDtypeStruct((rows, d), sequence.dtype),
        mesh=mesh,
        scratch_types=[
            pltpu.VMEM((_R, d), sequence.dtype),
            pltpu.VMEM((_R, d), sequence.dtype),
        ],
    )(seq2d, pe)
    return out2d.reshape(b, s, d)
